# baseline (speedup 1.0000x reference)
import functools

import jax
import jax.numpy as jnp
from jax import lax
from jax.experimental import pallas as pl
from jax.experimental.pallas import tpu as pltpu
from jax.experimental.pallas import tpu_sc as plsc

CHUNK = 128
K = 4
NBUF = 3


@functools.cache
def _build_gather(n_total, emb):
    info = plsc.get_sparse_core_info()
    num_workers = info.num_cores * info.num_subcores
    group = K * CHUNK
    assert n_total % (num_workers * group) == 0
    G = n_total // (num_workers * group)
    rows_per_worker = G * K
    assert G >= NBUF + 1

    mesh = plsc.VectorSubcoreMesh(core_axis_name="c", subcore_axis_name="s")

    @functools.partial(
        pl.kernel,
        mesh=mesh,
        out_type=jax.ShapeDtypeStruct((n_total, 2 * emb), jnp.float32),
        scratch_types=[
            pltpu.VMEM((NBUF, K, CHUNK), jnp.int32),
            pltpu.VMEM((NBUF, K * CHUNK, emb), jnp.float32),
        ]
        + [pltpu.SemaphoreType.DMA] * (3 * NBUF),
        compiler_params=pltpu.CompilerParams(use_tc_tiling_on_sc=False),
    )
    def gather(idx_hbm, table_hbm, out_hbm, idx_v, rows_v, *sems):
        isem = sems[0:NBUF]
        gsem = sems[NBUF:2 * NBUF]
        wsem = sems[2 * NBUF:3 * NBUF]
        wid = lax.axis_index("s") * info.num_cores + lax.axis_index("c")
        row0 = wid * rows_per_worker

        def idx_copy(p, s):
            return pltpu.make_async_copy(
                idx_hbm.at[pl.ds(row0 + p * K, K)], idx_v.at[s], isem[s])

        def gathers(p, s):
            return [
                pltpu.make_async_copy(
                    table_hbm.at[idx_v.at[s, j]],
                    rows_v.at[s, pl.ds(j * CHUNK, CHUNK)],
                    gsem[s])
                for j in range(K)
            ]

        def wr_copy(p, s):
            return pltpu.make_async_copy(
                rows_v.at[s],
                out_hbm.at[pl.ds((row0 + p * K) * CHUNK, K * CHUNK),
                           pl.ds(0, emb)],
                wsem[s])

        def fire(p, s, guard_rows):
            idx_copy(p, s).wait()
            if guard_rows:
                @pl.when(p >= NBUF)
                def _():
                    wr_copy(p - NBUF, s).wait()
            gs = gathers(p, s)
            for g in gs:
                g.start()
            return gs

        def drain(p, s):
            for g in gathers(p, s):
                g.wait()
            wr_copy(p, s).start()

        idx_copy(0, 0).start()
        fire(0, 0, guard_rows=False)
        idx_copy(1, 1).start()

        main_groups = G - 2
        iters = main_groups // NBUF

        def body(i, carry):
            for b in range(NBUF):
                p = 1 + i * NBUF + b
                s = (1 + b) % NBUF
                fire(p, s, guard_rows=True)
                idx_copy(p + 1, (s + 1) % NBUF).start()
                drain(p - 1, (s - 1) % NBUF)
            return carry

        lax.fori_loop(0, iters, body, 0)

        for p in range(1 + iters * NBUF, G):
            s = p % NBUF
            idx_copy(p, s).wait()
            if p >= NBUF:
                wr_copy(p - NBUF, s).wait()
            for g in gathers(p, s):
                g.start()
            if p + 1 < G:
                idx_copy(p + 1, (p + 1) % NBUF).start()
            drain(p - 1, (p - 1) % NBUF)

        drain(G - 1, (G - 1) % NBUF)
        for p in range(max(0, G - NBUF), G):
            wr_copy(p, p % NBUF).wait()

    return gather


VSTEP = 256
MAIN_STEPS = 122


@functools.cache
def _build_transpose(vocab, emb):
    info = plsc.get_sparse_core_info()
    num_workers = info.num_cores * info.num_subcores
    main_cols = num_workers * MAIN_STEPS * VSTEP
    cut = (vocab // VSTEP) * VSTEP
    peel_cols = list(range(main_cols, cut, VSTEP))
    tail_rows = (vocab - cut) // 2
    out_rows = cut // 2 + tail_rows

    mesh = plsc.VectorSubcoreMesh(core_axis_name="c", subcore_axis_name="s")

    @functools.partial(
        pl.kernel,
        mesh=mesh,
        out_type=jax.ShapeDtypeStruct((out_rows, 2 * emb), jnp.float32),
        scratch_types=[
            pltpu.VMEM((2, emb, VSTEP), jnp.float32),
            pltpu.VMEM((2, VSTEP // 2, 2 * emb), jnp.float32),
            pltpu.VMEM((tail_rows, 2 * emb), jnp.float32),
        ]
        + [pltpu.SemaphoreType.DMA] * 4,
        compiler_params=pltpu.CompilerParams(needs_layout_passes=False),
    )
    def transpose(tt_hbm, tail_hbm, out_hbm, buf, obuf, tbuf,
                  isem0, isem1, osem0, osem1):
        isem = (isem0, isem1)
        osem = (osem0, osem1)
        wid = lax.axis_index("s") * info.num_cores + lax.axis_index("c")
        col0 = wid * MAIN_STEPS * VSTEP

        lanes = lax.iota(jnp.int32, 16)
        c_vecs = [lanes + 16 * k for k in range(emb // 16)]

        def in_copy(i, s):
            off = pl.multiple_of(col0 + i * VSTEP, VSTEP)
            return pltpu.make_async_copy(
                tt_hbm.at[:, pl.ds(off, VSTEP)], buf.at[s], isem[s])

        def out_copy(i, s):
            off = pl.multiple_of((col0 + i * VSTEP) // 2, VSTEP // 2)
            return pltpu.make_async_copy(
                obuf.at[s], out_hbm.at[pl.ds(off, VSTEP // 2)], osem[s])

        def compute(s):
            s_vec = jnp.full((16,), s, dtype=jnp.int32)

            def qbody(q, carry):
                for h in range(2):
                    r_vec = jnp.full((16,), 2 * q + h, dtype=jnp.int32)
                    for k in range(emb // 16):
                        v = plsc.load_gather(buf, [s_vec, c_vecs[k], r_vec])
                        obuf[s, q, pl.ds(h * emb + 16 * k, 16)] = v
                return carry

            lax.fori_loop(0, VSTEP // 2, qbody, 0)

        in_copy(0, 0).start()

        def body(i2, carry):
            for s in range(2):
                i = i2 * 2 + s
                in_copy(i, s).wait()
                if s == 0:
                    in_copy(i + 1, 1).start()
                else:
                    @pl.when(i2 < MAIN_STEPS // 2 - 1)
                    def _():
                        in_copy(i + 1, 0).start()

                @pl.when(i2 >= 1)
                def _():
                    out_copy(i - 2, s).wait()
                compute(s)
                out_copy(i, s).start()
            return carry

        lax.fori_loop(0, MAIN_STEPS // 2, body, 0)
        out_copy(MAIN_STEPS - 2, 0).wait()
        out_copy(MAIN_STEPS - 1, 1).wait()

        for w, c0 in enumerate(peel_cols):
            @pl.when(wid == w)
            def _():
                pltpu.sync_copy(tt_hbm.at[:, pl.ds(c0, VSTEP)], buf.at[0])
                compute(0)
                pltpu.sync_copy(
                    obuf.at[0], out_hbm.at[pl.ds(c0 // 2, VSTEP // 2)])

        @pl.when(wid == len(peel_cols))
        def _():
            pltpu.sync_copy(tail_hbm, tbuf)
            pltpu.sync_copy(tbuf, out_hbm.at[pl.ds(cut // 2, tail_rows)])

    return transpose


def kernel(x, table):
    b, l = x.shape
    v, emb = table.shape
    n = b * l
    idx = x.reshape(n // CHUNK, CHUNK)
    cut = (v // VSTEP) * VSTEP
    tail = table[cut:].reshape((v - cut) // 2, 2 * emb)
    tdense = _build_transpose(v, emb)(table.T, tail).reshape(v, emb)
    out = _build_gather(n, emb)(idx, tdense)
    return out[:, :emb].reshape(b, l, emb)


# speedup vs baseline: 1.2899x; 1.2899x over previous
import functools

import jax
import jax.numpy as jnp
from jax import lax
from jax.experimental import pallas as pl
from jax.experimental.pallas import tpu as pltpu
from jax.experimental.pallas import tpu_sc as plsc

CHUNK = 128
K = 4
NBUF = 3


@functools.cache
def _build_gather(n_total, emb):
    info = plsc.get_sparse_core_info()
    num_workers = info.num_cores * info.num_subcores
    group = K * CHUNK
    assert n_total % (num_workers * group) == 0
    G = n_total // (num_workers * group)
    rows_per_worker = G * K
    assert G >= NBUF + 1

    mesh = plsc.VectorSubcoreMesh(core_axis_name="c", subcore_axis_name="s")

    @functools.partial(
        pl.kernel,
        mesh=mesh,
        out_type=jax.ShapeDtypeStruct((n_total, 2 * emb), jnp.float32),
        scratch_types=[
            pltpu.VMEM((NBUF, K, CHUNK), jnp.int32),
            pltpu.VMEM((NBUF, K * CHUNK, emb), jnp.float32),
        ]
        + [pltpu.SemaphoreType.DMA] * (3 * NBUF),
        compiler_params=pltpu.CompilerParams(use_tc_tiling_on_sc=False),
    )
    def gather(idx_hbm, table_hbm, out_hbm, idx_v, rows_v, *sems):
        isem = sems[0:NBUF]
        gsem = sems[NBUF:2 * NBUF]
        wsem = sems[2 * NBUF:3 * NBUF]
        wid = lax.axis_index("s") * info.num_cores + lax.axis_index("c")
        row0 = wid * rows_per_worker

        def idx_copy(p, s):
            return pltpu.make_async_copy(
                idx_hbm.at[pl.ds(row0 + p * K, K)], idx_v.at[s], isem[s])

        def gathers(p, s):
            return [
                pltpu.make_async_copy(
                    table_hbm.at[idx_v.at[s, j]],
                    rows_v.at[s, pl.ds(j * CHUNK, CHUNK)],
                    gsem[s])
                for j in range(K)
            ]

        def wr_copy(p, s):
            return pltpu.make_async_copy(
                rows_v.at[s],
                out_hbm.at[pl.ds((row0 + p * K) * CHUNK, K * CHUNK),
                           pl.ds(0, emb)],
                wsem[s])

        def fire(p, s, guard_rows):
            idx_copy(p, s).wait()
            if guard_rows:
                @pl.when(p >= NBUF)
                def _():
                    wr_copy(p - NBUF, s).wait()
            gs = gathers(p, s)
            for g in gs:
                g.start()
            return gs

        def drain(p, s):
            for g in gathers(p, s):
                g.wait()
            wr_copy(p, s).start()

        idx_copy(0, 0).start()
        fire(0, 0, guard_rows=False)
        idx_copy(1, 1).start()

        main_groups = G - 2
        iters = main_groups // NBUF

        def body(i, carry):
            for b in range(NBUF):
                p = 1 + i * NBUF + b
                s = (1 + b) % NBUF
                fire(p, s, guard_rows=True)
                idx_copy(p + 1, (s + 1) % NBUF).start()
                drain(p - 1, (s - 1) % NBUF)
            return carry

        lax.fori_loop(0, iters, body, 0)

        for p in range(1 + iters * NBUF, G):
            s = p % NBUF
            idx_copy(p, s).wait()
            if p >= NBUF:
                wr_copy(p - NBUF, s).wait()
            for g in gathers(p, s):
                g.start()
            if p + 1 < G:
                idx_copy(p + 1, (p + 1) % NBUF).start()
            drain(p - 1, (p - 1) % NBUF)

        drain(G - 1, (G - 1) % NBUF)
        for p in range(max(0, G - NBUF), G):
            wr_copy(p, p % NBUF).wait()

    return gather


VSTEP = 256
MAIN_STEPS = 122


@functools.cache
def _build_transpose(vocab, emb):
    info = plsc.get_sparse_core_info()
    num_workers = info.num_cores * info.num_subcores
    main_cols = num_workers * MAIN_STEPS * VSTEP
    cut = (vocab // VSTEP) * VSTEP
    peel_cols = list(range(main_cols, cut, VSTEP))
    tail_rows = (vocab - cut) // 2
    out_rows = cut // 2 + tail_rows

    mesh = plsc.VectorSubcoreMesh(core_axis_name="c", subcore_axis_name="s")

    @functools.partial(
        pl.kernel,
        mesh=mesh,
        out_type=jax.ShapeDtypeStruct((out_rows, 2 * emb), jnp.float32),
        scratch_types=[
            pltpu.VMEM((2, emb, VSTEP), jnp.float32),
            pltpu.VMEM((2, VSTEP // 2, 2 * emb), jnp.float32),
            pltpu.VMEM((tail_rows, 2 * emb), jnp.float32),
        ]
        + [pltpu.SemaphoreType.DMA] * 4,
        compiler_params=pltpu.CompilerParams(needs_layout_passes=False),
    )
    def transpose(tt_hbm, tail_hbm, out_hbm, buf, obuf, tbuf,
                  isem0, isem1, osem0, osem1):
        isem = (isem0, isem1)
        osem = (osem0, osem1)
        wid = lax.axis_index("s") * info.num_cores + lax.axis_index("c")
        col0 = wid * MAIN_STEPS * VSTEP

        lanes = lax.iota(jnp.int32, 16)
        c_vecs = [lanes + 16 * k for k in range(emb // 16)]

        def in_copy(i, s):
            off = pl.multiple_of(col0 + i * VSTEP, VSTEP)
            return pltpu.make_async_copy(
                tt_hbm.at[:, pl.ds(off, VSTEP)], buf.at[s], isem[s])

        def out_copy(i, s):
            off = pl.multiple_of((col0 + i * VSTEP) // 2, VSTEP // 2)
            return pltpu.make_async_copy(
                obuf.at[s], out_hbm.at[pl.ds(off, VSTEP // 2)], osem[s])

        def compute(s):
            s_vec = jnp.full((16,), s, dtype=jnp.int32)
            unroll = 8

            def qbody(qq, carry):
                r_base = jnp.full((16,), 2 * unroll * qq, dtype=jnp.int32)
                vs = []
                for u in range(unroll):
                    for h in range(2):
                        r_vec = r_base + (2 * u + h)
                        for k in range(emb // 16):
                            vs.append(plsc.load_gather(
                                buf, [s_vec, c_vecs[k], r_vec]))
                i = 0
                for u in range(unroll):
                    q = unroll * qq + u
                    for h in range(2):
                        for k in range(emb // 16):
                            obuf[s, q, pl.ds(h * emb + 16 * k, 16)] = vs[i]
                            i += 1
                return carry

            lax.fori_loop(0, VSTEP // 2 // unroll, qbody, 0)

        in_copy(0, 0).start()

        def body(i2, carry):
            for s in range(2):
                i = i2 * 2 + s
                in_copy(i, s).wait()
                if s == 0:
                    in_copy(i + 1, 1).start()
                else:
                    @pl.when(i2 < MAIN_STEPS // 2 - 1)
                    def _():
                        in_copy(i + 1, 0).start()

                @pl.when(i2 >= 1)
                def _():
                    out_copy(i - 2, s).wait()
                compute(s)
                out_copy(i, s).start()
            return carry

        lax.fori_loop(0, MAIN_STEPS // 2, body, 0)
        out_copy(MAIN_STEPS - 2, 0).wait()
        out_copy(MAIN_STEPS - 1, 1).wait()

        for w, c0 in enumerate(peel_cols):
            @pl.when(wid == w)
            def _():
                pltpu.sync_copy(tt_hbm.at[:, pl.ds(c0, VSTEP)], buf.at[0])
                compute(0)
                pltpu.sync_copy(
                    obuf.at[0], out_hbm.at[pl.ds(c0 // 2, VSTEP // 2)])

        @pl.when(wid == len(peel_cols))
        def _():
            pltpu.sync_copy(tail_hbm, tbuf)
            pltpu.sync_copy(tbuf, out_hbm.at[pl.ds(cut // 2, tail_rows)])

    return transpose


def kernel(x, table):
    b, l = x.shape
    v, emb = table.shape
    n = b * l
    idx = x.reshape(n // CHUNK, CHUNK)
    cut = (v // VSTEP) * VSTEP
    tail = table[cut:].reshape((v - cut) // 2, 2 * emb)
    tdense = _build_transpose(v, emb)(table.T, tail).reshape(v, emb)
    out = _build_gather(n, emb)(idx, tdense)
    return out[:, :emb].reshape(b, l, emb)


# speedup vs baseline: 1.2977x; 1.0060x over previous
import functools

import jax
import jax.numpy as jnp
from jax import lax
from jax.experimental import pallas as pl
from jax.experimental.pallas import tpu as pltpu
from jax.experimental.pallas import tpu_sc as plsc

CHUNK = 128
K = 4
NBUF = 3


@functools.cache
def _build_gather(n_total, emb):
    info = plsc.get_sparse_core_info()
    num_workers = info.num_cores * info.num_subcores
    group = K * CHUNK
    assert n_total % (num_workers * group) == 0
    G = n_total // (num_workers * group)
    rows_per_worker = G * K
    assert G >= NBUF + 1

    mesh = plsc.VectorSubcoreMesh(core_axis_name="c", subcore_axis_name="s")

    @functools.partial(
        pl.kernel,
        mesh=mesh,
        out_type=jax.ShapeDtypeStruct((n_total, 2 * emb), jnp.float32),
        scratch_types=[
            pltpu.VMEM((NBUF, K, CHUNK), jnp.int32),
            pltpu.VMEM((NBUF, K * CHUNK, emb), jnp.float32),
        ]
        + [pltpu.SemaphoreType.DMA] * (3 * NBUF),
        compiler_params=pltpu.CompilerParams(use_tc_tiling_on_sc=False),
    )
    def gather(idx_hbm, table_hbm, out_hbm, idx_v, rows_v, *sems):
        isem = sems[0:NBUF]
        gsem = sems[NBUF:2 * NBUF]
        wsem = sems[2 * NBUF:3 * NBUF]
        wid = lax.axis_index("s") * info.num_cores + lax.axis_index("c")
        row0 = wid * rows_per_worker

        def idx_copy(p, s):
            return pltpu.make_async_copy(
                idx_hbm.at[pl.ds(row0 + p * K, K)], idx_v.at[s], isem[s])

        def gathers(p, s):
            return [
                pltpu.make_async_copy(
                    table_hbm.at[idx_v.at[s, j]],
                    rows_v.at[s, pl.ds(j * CHUNK, CHUNK)],
                    gsem[s])
                for j in range(K)
            ]

        def wr_copy(p, s):
            return pltpu.make_async_copy(
                rows_v.at[s],
                out_hbm.at[pl.ds((row0 + p * K) * CHUNK, K * CHUNK),
                           pl.ds(0, emb)],
                wsem[s])

        def fire(p, s, guard_rows):
            idx_copy(p, s).wait()
            if guard_rows:
                @pl.when(p >= NBUF)
                def _():
                    wr_copy(p - NBUF, s).wait()
            gs = gathers(p, s)
            for g in gs:
                g.start()
            return gs

        def drain(p, s):
            for g in gathers(p, s):
                g.wait()
            wr_copy(p, s).start()

        idx_copy(0, 0).start()
        fire(0, 0, guard_rows=False)
        idx_copy(1, 1).start()

        main_groups = G - 2
        iters = main_groups // NBUF

        def body(i, carry):
            for b in range(NBUF):
                p = 1 + i * NBUF + b
                s = (1 + b) % NBUF
                fire(p, s, guard_rows=True)
                idx_copy(p + 1, (s + 1) % NBUF).start()
                drain(p - 1, (s - 1) % NBUF)
            return carry

        lax.fori_loop(0, iters, body, 0)

        for p in range(1 + iters * NBUF, G):
            s = p % NBUF
            idx_copy(p, s).wait()
            if p >= NBUF:
                wr_copy(p - NBUF, s).wait()
            for g in gathers(p, s):
                g.start()
            if p + 1 < G:
                idx_copy(p + 1, (p + 1) % NBUF).start()
            drain(p - 1, (p - 1) % NBUF)

        drain(G - 1, (G - 1) % NBUF)
        for p in range(max(0, G - NBUF), G):
            wr_copy(p, p % NBUF).wait()

    return gather


VSTEP = 256
MAIN_STEPS = 122


@functools.cache
def _build_transpose(vocab, emb):
    info = plsc.get_sparse_core_info()
    num_workers = info.num_cores * info.num_subcores
    main_cols = num_workers * MAIN_STEPS * VSTEP
    cut = (vocab // VSTEP) * VSTEP
    peel_cols = list(range(main_cols, cut, VSTEP))
    tail_rows = (vocab - cut) // 2
    out_rows = cut // 2 + tail_rows

    mesh = plsc.VectorSubcoreMesh(core_axis_name="c", subcore_axis_name="s")

    @functools.partial(
        pl.kernel,
        mesh=mesh,
        out_type=jax.ShapeDtypeStruct((out_rows, 2 * emb), jnp.float32),
        scratch_types=[
            pltpu.VMEM((2, emb, VSTEP + 1), jnp.float32),
            pltpu.VMEM((2, VSTEP // 2, 2 * emb), jnp.float32),
            pltpu.VMEM((tail_rows, 2 * emb), jnp.float32),
        ]
        + [pltpu.SemaphoreType.DMA] * 4,
        compiler_params=pltpu.CompilerParams(needs_layout_passes=False),
    )
    def transpose(tt_hbm, tail_hbm, out_hbm, buf, obuf, tbuf,
                  isem0, isem1, osem0, osem1):
        isem = (isem0, isem1)
        osem = (osem0, osem1)
        wid = lax.axis_index("s") * info.num_cores + lax.axis_index("c")
        col0 = wid * MAIN_STEPS * VSTEP

        lanes = lax.iota(jnp.int32, 16)
        c_vecs = [lanes + 16 * k for k in range(emb // 16)]

        def in_copy(i, s):
            off = pl.multiple_of(col0 + i * VSTEP, VSTEP)
            return pltpu.make_async_copy(
                tt_hbm.at[:, pl.ds(off, VSTEP)],
                buf.at[s, :, pl.ds(0, VSTEP)], isem[s])

        def out_copy(i, s):
            off = pl.multiple_of((col0 + i * VSTEP) // 2, VSTEP // 2)
            return pltpu.make_async_copy(
                obuf.at[s], out_hbm.at[pl.ds(off, VSTEP // 2)], osem[s])

        def compute(s):
            s_vec = jnp.full((16,), s, dtype=jnp.int32)
            unroll = 8

            def qbody(qq, carry):
                r_base = jnp.full((16,), 2 * unroll * qq, dtype=jnp.int32)
                vs = []
                for u in range(unroll):
                    for h in range(2):
                        r_vec = r_base + (2 * u + h)
                        for k in range(emb // 16):
                            vs.append(plsc.load_gather(
                                buf, [s_vec, c_vecs[k], r_vec]))
                i = 0
                for u in range(unroll):
                    q = unroll * qq + u
                    for h in range(2):
                        for k in range(emb // 16):
                            obuf[s, q, pl.ds(h * emb + 16 * k, 16)] = vs[i]
                            i += 1
                return carry

            lax.fori_loop(0, VSTEP // 2 // unroll, qbody, 0)

        in_copy(0, 0).start()

        def body(i2, carry):
            for s in range(2):
                i = i2 * 2 + s
                in_copy(i, s).wait()
                if s == 0:
                    in_copy(i + 1, 1).start()
                else:
                    @pl.when(i2 < MAIN_STEPS // 2 - 1)
                    def _():
                        in_copy(i + 1, 0).start()

                @pl.when(i2 >= 1)
                def _():
                    out_copy(i - 2, s).wait()
                compute(s)
                out_copy(i, s).start()
            return carry

        lax.fori_loop(0, MAIN_STEPS // 2, body, 0)
        out_copy(MAIN_STEPS - 2, 0).wait()
        out_copy(MAIN_STEPS - 1, 1).wait()

        for w, c0 in enumerate(peel_cols):
            @pl.when(wid == w)
            def _():
                pltpu.sync_copy(tt_hbm.at[:, pl.ds(c0, VSTEP)],
                                buf.at[0, :, pl.ds(0, VSTEP)])
                compute(0)
                pltpu.sync_copy(
                    obuf.at[0], out_hbm.at[pl.ds(c0 // 2, VSTEP // 2)])

        @pl.when(wid == len(peel_cols))
        def _():
            pltpu.sync_copy(tail_hbm, tbuf)
            pltpu.sync_copy(tbuf, out_hbm.at[pl.ds(cut // 2, tail_rows)])

    return transpose


def kernel(x, table):
    b, l = x.shape
    v, emb = table.shape
    n = b * l
    idx = x.reshape(n // CHUNK, CHUNK)
    cut = (v // VSTEP) * VSTEP
    tail = table[cut:].reshape((v - cut) // 2, 2 * emb)
    tdense = _build_transpose(v, emb)(table.T, tail).reshape(v, emb)
    out = _build_gather(n, emb)(idx, tdense)
    return out[:, :emb].reshape(b, l, emb)


# speedup vs baseline: 1.9747x; 1.5216x over previous
import functools

import jax
import jax.numpy as jnp
from jax import lax
from jax.experimental import pallas as pl
from jax.experimental.pallas import tpu as pltpu
from jax.experimental.pallas import tpu_sc as plsc

CHUNK = 128
K = 4
NBUF = 3


@functools.cache
def _build_gather(n_total, emb):
    info = plsc.get_sparse_core_info()
    num_workers = info.num_cores * info.num_subcores
    group = K * CHUNK
    assert n_total % (num_workers * group) == 0
    G = n_total // (num_workers * group)
    rows_per_worker = G * K
    assert G >= NBUF + 1

    mesh = plsc.VectorSubcoreMesh(core_axis_name="c", subcore_axis_name="s")

    @functools.partial(
        pl.kernel,
        mesh=mesh,
        out_type=jax.ShapeDtypeStruct((n_total, 2 * emb), jnp.float32),
        scratch_types=[
            pltpu.VMEM((NBUF, K, CHUNK), jnp.int32),
            pltpu.VMEM((NBUF, K * CHUNK, emb), jnp.float32),
        ]
        + [pltpu.SemaphoreType.DMA] * (3 * NBUF),
        compiler_params=pltpu.CompilerParams(use_tc_tiling_on_sc=False),
    )
    def gather(idx_hbm, table_hbm, out_hbm, idx_v, rows_v, *sems):
        isem = sems[0:NBUF]
        gsem = sems[NBUF:2 * NBUF]
        wsem = sems[2 * NBUF:3 * NBUF]
        wid = lax.axis_index("s") * info.num_cores + lax.axis_index("c")
        row0 = wid * rows_per_worker

        def idx_copy(p, s):
            return pltpu.make_async_copy(
                idx_hbm.at[pl.ds(row0 + p * K, K)], idx_v.at[s], isem[s])

        def gathers(p, s):
            return [
                pltpu.make_async_copy(
                    table_hbm.at[idx_v.at[s, j]],
                    rows_v.at[s, pl.ds(j * CHUNK, CHUNK)],
                    gsem[s])
                for j in range(K)
            ]

        def wr_copy(p, s):
            return pltpu.make_async_copy(
                rows_v.at[s],
                out_hbm.at[pl.ds((row0 + p * K) * CHUNK, K * CHUNK),
                           pl.ds(0, emb)],
                wsem[s])

        def fire(p, s, guard_rows):
            idx_copy(p, s).wait()
            if guard_rows:
                @pl.when(p >= NBUF)
                def _():
                    wr_copy(p - NBUF, s).wait()
            gs = gathers(p, s)
            for g in gs:
                g.start()
            return gs

        def drain(p, s):
            for g in gathers(p, s):
                g.wait()
            wr_copy(p, s).start()

        idx_copy(0, 0).start()
        fire(0, 0, guard_rows=False)
        idx_copy(1, 1).start()

        main_groups = G - 2
        iters = main_groups // NBUF

        def body(i, carry):
            for b in range(NBUF):
                p = 1 + i * NBUF + b
                s = (1 + b) % NBUF
                fire(p, s, guard_rows=True)
                idx_copy(p + 1, (s + 1) % NBUF).start()
                drain(p - 1, (s - 1) % NBUF)
            return carry

        lax.fori_loop(0, iters, body, 0)

        for p in range(1 + iters * NBUF, G):
            s = p % NBUF
            idx_copy(p, s).wait()
            if p >= NBUF:
                wr_copy(p - NBUF, s).wait()
            for g in gathers(p, s):
                g.start()
            if p + 1 < G:
                idx_copy(p + 1, (p + 1) % NBUF).start()
            drain(p - 1, (p - 1) % NBUF)

        drain(G - 1, (G - 1) % NBUF)
        for p in range(max(0, G - NBUF), G):
            wr_copy(p, p % NBUF).wait()

    return gather


def kernel(x, table):
    b, l = x.shape
    _, emb = table.shape
    n = b * l
    idx = x.reshape(n // CHUNK, CHUNK)
    out = _build_gather(n, emb)(idx, table)
    return out[:, :emb].reshape(b, l, emb)
